# baseline (speedup 1.0000x reference)
import functools

import jax
import jax.numpy as jnp
from jax import lax
from jax.experimental import pallas as pl
from jax.experimental.pallas import tpu as pltpu
from jax.experimental.pallas import tpu_sc as plsc

_C = 128
_NW = 32
_AR = 400


def _make_sc_convert(n, d):
    nblk = n // _AR
    bpw = nblk // _NW
    tail = nblk - bpw * _NW
    mesh = plsc.VectorSubcoreMesh(core_axis_name="c", subcore_axis_name="s")

    @functools.partial(
        pl.kernel,
        mesh=mesh,
        compiler_params=pltpu.CompilerParams(
            use_tc_tiling_on_sc=True, needs_layout_passes=False
        ),
        out_type=jax.ShapeDtypeStruct((n * d // 2,), jnp.int32),
        scratch_types=[
            pltpu.VMEM((2, _AR, d), jnp.float32),
            pltpu.VMEM((_AR * d // 2,), jnp.int32),
            pltpu.VMEM((_AR * d // 2,), jnp.int32),
            pltpu.SemaphoreType.DMA,
            pltpu.SemaphoreType.DMA,
        ],
    )
    def sc_convert(emb_hbm, out_hbm, vf, vb0, vb1, semi0, semi1):
        wid = lax.axis_index("s") * 2 + lax.axis_index("c")
        sems = (semi0, semi1)
        vbs = (vb0, vb1)

        def fetch(blk, b):
            r0 = pl.multiple_of(blk * _AR, _AR)
            for bb in range(2):
                @pl.when(b == bb)
                def _():
                    pltpu.async_copy(emb_hbm.at[pl.ds(r0, _AR)], vf.at[bb], sems[bb])

        def wait_fetch(b):
            for bb in range(2):
                @pl.when(b == bb)
                def _():
                    pltpu.make_async_copy(
                        emb_hbm.at[pl.ds(0, _AR)], vf.at[bb], sems[bb]
                    ).wait()

        def convert(bb):
            def row(j, c2):
                for h in range(d // 32):
                    a = vf[bb, j, pl.ds(h * 32, 16)]
                    x = vf[bb, j, pl.ds(h * 32 + 16, 16)]
                    p = plsc.pack(a, x, format=plsc.PackFormat.INTERLEAVED)
                    vbs[bb][pl.ds(j * (d // 2) + h * 16, 16)] = plsc.bitcast(
                        p, jnp.int32
                    )
                return c2

            lax.fori_loop(0, _AR, row, 0, unroll=False)

        nmine = bpw + jnp.where(wid < tail, 1, 0)

        def blk_of(i):
            return jnp.where(i < bpw, wid * bpw + i, _NW * bpw + wid)

        fetch(blk_of(0), 0)

        def body(i, carry):
            b = i % 2

            @pl.when(i + 1 < nmine)
            def _():
                fetch(blk_of(i + 1), 1 - b)

            wait_fetch(b)
            for bb in range(2):
                @pl.when(b == bb)
                def _():
                    convert(bb)
                    w = _AR * d // 2
                    o0 = pl.multiple_of(blk_of(i) * w, w)
                    pltpu.sync_copy(vbs[bb], out_hbm.at[pl.ds(o0, w)])
            return carry

        lax.fori_loop(0, nmine, body, 0, unroll=False)

    return sc_convert


def _make_sc_norms(d, ng, arity, npos_chunks):
    nchunks = ng // _C
    cpw = nchunks // _NW
    cw = _C * arity
    mesh = plsc.VectorSubcoreMesh(core_axis_name="c", subcore_axis_name="s")

    @functools.partial(
        pl.kernel,
        mesh=mesh,
        compiler_params=pltpu.CompilerParams(
            use_tc_tiling_on_sc=False, needs_layout_passes=False
        ),
        out_type=jax.ShapeDtypeStruct((ng, 16), jnp.float32),
        scratch_types=[
            pltpu.VMEM((2, cw), jnp.int32),
            pltpu.VMEM((2, arity, _C), jnp.int32),
            pltpu.VMEM((2, _C, d), jnp.bfloat16),
            pltpu.VMEM((_C, 16), jnp.float32),
            pltpu.SemaphoreType.DMA,
            pltpu.SemaphoreType.DMA,
        ],
    )
    def sc_norms(
        emb_hbm, xpos_hbm, xneg_hbm, out_hbm, raw_v, idx_v, acc_v, norms_v, sem0, sem1
    ):
        wid = lax.axis_index("s") * 2 + lax.axis_index("c")
        lane = jnp.arange(16, dtype=jnp.int32)
        zb = jnp.zeros((32,), jnp.bfloat16)

        def stage_fire(gci, b, sem):
            @pl.when(gci < npos_chunks)
            def _():
                r0 = pl.multiple_of(gci * cw, cw)
                pltpu.sync_copy(xpos_hbm.at[pl.ds(r0, cw)], raw_v.at[b])

            @pl.when(gci >= npos_chunks)
            def _():
                r0 = pl.multiple_of((gci - npos_chunks) * cw, cw)
                pltpu.sync_copy(xneg_hbm.at[pl.ds(r0, cw)], raw_v.at[b])

            lane_a = lane * arity

            def tr_body(k, c2):
                for jb in range(_C // 16):
                    pos = (jb * 16 * arity + k) + lane_a
                    g = plsc.load_gather(raw_v.at[b], [pos])
                    idx_v[b, k, pl.ds(jb * 16, 16)] = g
                return c2

            lax.fori_loop(0, arity, tr_body, 0, unroll=False)
            for k in range(arity):
                pltpu.async_copy(emb_hbm.at[idx_v.at[b, k]], acc_v.at[b], sem, add=True)

        def drain(b, sem):
            for _ in range(arity):
                pltpu.make_async_copy(
                    emb_hbm.at[idx_v.at[b, 0]], acc_v.at[b], sem
                ).wait()

        def compute_out(gci, b):
            def grp_body(j, carry2):
                s = jnp.zeros((16,), jnp.float32)
                for c in range(d // 32):
                    ab = acc_v[b, j, pl.ds(c * 32, 32)]
                    acc_v[b, j, pl.ds(c * 32, 32)] = zb
                    x, y = plsc.unpack(ab, format=plsc.PackFormat.INTERLEAVED)
                    s = s + x * x + y * y
                norms_v[j, pl.ds(0, 16)] = s
                return carry2

            lax.fori_loop(0, _C, grp_body, 0, unroll=False)
            o0 = pl.multiple_of(gci * _C, _C)
            pltpu.sync_copy(norms_v, out_hbm.at[pl.ds(o0, _C), :])

        def zero_body(j, carry2):
            for b in range(2):
                for c in range(d // 32):
                    acc_v[b, j, pl.ds(c * 32, 32)] = zb
            return carry2

        lax.fori_loop(0, _C, zero_body, 0, unroll=False)

        base = wid * cpw
        stage_fire(base, 0, sem0)

        def pipe_body(h, carry):
            c0 = base + 2 * h
            stage_fire(c0 + 1, 1, sem1)
            drain(0, sem0)
            compute_out(c0, 0)

            @pl.when(2 * h + 2 < cpw)
            def _():
                stage_fire(c0 + 2, 0, sem0)

            drain(1, sem1)
            compute_out(c0 + 1, 1)
            return carry

        lax.fori_loop(0, cpw // 2, pipe_body, 0, unroll=False)

    return sc_norms


def _make_score(ng, batch):
    nrows = ng * 16 // 128
    rows_pos = batch // 8

    def score_body(part_ref, out_ref):
        x = part_ref[...]
        l = lax.broadcasted_iota(jnp.int32, (128, 8), 0)
        k = lax.broadcasted_iota(jnp.int32, (128, 8), 1)
        m = (l // 16 == k).astype(jnp.float32)
        y = jnp.dot(x, m, precision=lax.Precision.HIGHEST)
        rows = lax.broadcasted_iota(jnp.int32, (nrows, 8), 0)
        v = jnp.where(rows < rows_pos, y, 1.0 / y)
        out_ref[0, 0] = jnp.sum(jnp.log(jnp.tanh(v))) / batch

    return pl.pallas_call(
        score_body,
        out_shape=jax.ShapeDtypeStruct((1, 1), jnp.float32),
        out_specs=pl.BlockSpec(memory_space=pltpu.SMEM),
    )


def kernel(x_pos, x_neg, emb):
    batch, arity = x_pos.shape
    num_neg = x_neg.shape[1]
    n, d = emb.shape
    ng = batch * (1 + num_neg)
    assert ng % (_C * _NW) == 0 and d % 32 == 0 and batch % _C == 0

    emb_i = _make_sc_convert(n, d)(emb)
    table = jax.lax.bitcast_convert_type(emb_i, jnp.bfloat16).reshape(n, d)
    xp1 = x_pos.reshape(batch * arity)
    xn1 = x_neg.reshape(batch * num_neg * arity)

    part = _make_sc_norms(d, ng, arity, batch // _C)(table, xp1, xn1)
    score = _make_score(ng, batch)(part.reshape(ng * 16 // 128, 128))
    return score[0, 0]


# speedup vs baseline: 3.5433x; 3.5433x over previous
import functools

import jax
import jax.numpy as jnp
from jax import lax
from jax.experimental import pallas as pl
from jax.experimental.pallas import tpu as pltpu
from jax.experimental.pallas import tpu_sc as plsc

_C = 128
_NW = 32


def _make_sc_norms(d, ng, arity, npos_chunks):
    nchunks = ng // _C
    cpw = nchunks // _NW
    cw = _C * arity
    h = d // 2
    mesh = plsc.VectorSubcoreMesh(core_axis_name="c", subcore_axis_name="s")

    @functools.partial(
        pl.kernel,
        mesh=mesh,
        compiler_params=pltpu.CompilerParams(
            use_tc_tiling_on_sc=False, needs_layout_passes=False
        ),
        out_type=jax.ShapeDtypeStruct((ng, 16), jnp.float32),
        scratch_types=[
            pltpu.VMEM((2, cw), jnp.int32),
            pltpu.VMEM((2, 2, arity, _C), jnp.int32),
            pltpu.VMEM((2, 2, _C, h), jnp.float32),
            pltpu.VMEM((_C, 16), jnp.float32),
            pltpu.SemaphoreType.DMA,
            pltpu.SemaphoreType.DMA,
        ],
    )
    def sc_norms(
        tbl_hbm, xpos_hbm, xneg_hbm, out_hbm, raw_v, idx_v, acc_v, norms_v, sem0, sem1
    ):
        wid = lax.axis_index("s") * 2 + lax.axis_index("c")
        lane = jnp.arange(16, dtype=jnp.int32)
        zv = jnp.zeros((16,), jnp.float32)

        def stage_fire(gci, b, sem):
            @pl.when(gci < npos_chunks)
            def _():
                r0 = pl.multiple_of(gci * cw, cw)
                pltpu.sync_copy(xpos_hbm.at[pl.ds(r0, cw)], raw_v.at[b])

            @pl.when(gci >= npos_chunks)
            def _():
                r0 = pl.multiple_of((gci - npos_chunks) * cw, cw)
                pltpu.sync_copy(xneg_hbm.at[pl.ds(r0, cw)], raw_v.at[b])

            lane_a = lane * arity

            def tr_body(k, c2):
                for jb in range(_C // 16):
                    pos = (jb * 16 * arity + k) + lane_a
                    g = plsc.load_gather(raw_v.at[b], [pos])
                    ga = g + g
                    idx_v[b, 0, k, pl.ds(jb * 16, 16)] = ga
                    idx_v[b, 1, k, pl.ds(jb * 16, 16)] = ga + 1
                return c2

            lax.fori_loop(0, arity, tr_body, 0, unroll=False)
            for half in range(2):
                for k in range(arity):
                    pltpu.async_copy(
                        tbl_hbm.at[idx_v.at[b, half, k]],
                        acc_v.at[b, half],
                        sem,
                        add=True,
                    )

        def drain(b, sem):
            for _ in range(2 * arity):
                pltpu.make_async_copy(
                    tbl_hbm.at[idx_v.at[b, 0, 0]], acc_v.at[b, 0], sem
                ).wait()

        def compute_out(gci, b):
            def grp_body(j, carry2):
                s = zv
                for half in range(2):
                    for c in range(h // 16):
                        v = acc_v[b, half, j, pl.ds(c * 16, 16)]
                        acc_v[b, half, j, pl.ds(c * 16, 16)] = zv
                        s = s + v * v
                norms_v[j, pl.ds(0, 16)] = s
                return carry2

            lax.fori_loop(0, _C, grp_body, 0, unroll=False)
            o0 = pl.multiple_of(gci * _C, _C)
            pltpu.sync_copy(norms_v, out_hbm.at[pl.ds(o0, _C), :])

        def zero_body(j, carry2):
            for b in range(2):
                for half in range(2):
                    for c in range(h // 16):
                        acc_v[b, half, j, pl.ds(c * 16, 16)] = zv
            return carry2

        lax.fori_loop(0, _C, zero_body, 0, unroll=False)

        base = wid * cpw
        stage_fire(base, 0, sem0)

        def pipe_body(hh, carry):
            c0 = base + 2 * hh
            stage_fire(c0 + 1, 1, sem1)
            drain(0, sem0)
            compute_out(c0, 0)

            @pl.when(2 * hh + 2 < cpw)
            def _():
                stage_fire(c0 + 2, 0, sem0)

            drain(1, sem1)
            compute_out(c0 + 1, 1)
            return carry

        lax.fori_loop(0, cpw // 2, pipe_body, 0, unroll=False)

    return sc_norms


def _make_score(ng, batch):
    nrows = ng * 16 // 128
    rows_pos = batch // 8

    def score_body(part_ref, out_ref):
        x = part_ref[...]
        l = lax.broadcasted_iota(jnp.int32, (128, 8), 0)
        k = lax.broadcasted_iota(jnp.int32, (128, 8), 1)
        m = (l // 16 == k).astype(jnp.float32)
        y = jnp.dot(x, m, precision=lax.Precision.HIGHEST)
        rows = lax.broadcasted_iota(jnp.int32, (nrows, 8), 0)
        v = jnp.where(rows < rows_pos, y, 1.0 / y)
        out_ref[0, 0] = jnp.sum(jnp.log(jnp.tanh(v))) / batch

    return pl.pallas_call(
        score_body,
        out_shape=jax.ShapeDtypeStruct((1, 1), jnp.float32),
        out_specs=pl.BlockSpec(memory_space=pltpu.SMEM),
    )


def kernel(x_pos, x_neg, emb):
    batch, arity = x_pos.shape
    num_neg = x_neg.shape[1]
    n, d = emb.shape
    ng = batch * (1 + num_neg)
    assert ng % (_C * _NW) == 0 and d % 32 == 0 and batch % _C == 0

    tbl = emb.reshape(2 * n, d // 2)
    xp1 = x_pos.reshape(batch * arity)
    xn1 = x_neg.reshape(batch * num_neg * arity)

    part = _make_sc_norms(d, ng, arity, batch // _C)(tbl, xp1, xn1)
    score = _make_score(ng, batch)(part.reshape(ng * 16 // 128, 128))
    return score[0, 0]
